# Initial kernel scaffold; baseline (speedup 1.0000x reference)
import jax
import jax.numpy as jnp
from jax.experimental import pallas as pl


def kernel(t1x, t2_embed1, t2_embed2, min_feature_embed, delta_feature_embed):
    raise NotImplementedError("write your pallas kernel here")


# speedup vs baseline: 4.1305x; 4.1305x over previous
import jax
import jax.numpy as jnp
from jax import lax
from jax.experimental import pallas as pl
from jax.experimental.pallas import tpu as pltpu
from jax.experimental.pallas import tpu_sc as plsc

MIN_VAR_, MIN_MEAN_ = 5.5, 4.5
DELTA_VAR_, DELTA_MEAN_ = 0.95, 1.05
EPS_ = 1e-8
LN2_ = 0.6931471805599453
SQRT2_ = 1.4142135623730951
L_ = 16


def _vlog(x):
    bits = lax.bitcast_convert_type(x, jnp.int32)
    e = (bits >> 23) - 127
    m = lax.bitcast_convert_type((bits & 0x007FFFFF) | 0x3F800000, jnp.float32)
    big = m >= SQRT2_
    m = jnp.where(big, m * 0.5, m)
    e = e + jnp.where(big, 1, 0)
    s = (m - 1.0) / (m + 1.0)
    s2 = s * s
    p = s * (2.0 + s2 * (2.0 / 3.0 + s2 * (0.4 + s2 * (2.0 / 7.0 + s2 * (2.0 / 9.0)))))
    return e.astype(jnp.float32) * LN2_ + p


def _make_sc_call(B):
    info = plsc.get_sparse_core_info()
    NC, NS = info.num_cores, info.num_subcores
    NW = NC * NS
    assert B % (NW * L_) == 0
    BPW = B // NW
    STEPS = BPW // L_

    mesh = plsc.VectorSubcoreMesh(core_axis_name="c", subcore_axis_name="s")

    def body(t1_h, e_h, idx_h, tmin_h, tdel_h, pos_h, neg_h,
             x0_v, x1_v, e0_v, e1_v, idx_v, tmin_v, tdel_v, pos_v, neg_v):
        wid = lax.axis_index("s") * NC + lax.axis_index("c")
        base = wid * BPW

        pltpu.sync_copy(t1_h.at[pl.ds(base, BPW)], x0_v)
        pltpu.sync_copy(t1_h.at[pl.ds(B + base, BPW)], x1_v)
        pltpu.sync_copy(e_h.at[pl.ds(base, BPW)], e0_v)
        pltpu.sync_copy(e_h.at[pl.ds(B + base, BPW)], e1_v)
        pltpu.sync_copy(idx_h.at[pl.ds(base, BPW)], idx_v)
        pltpu.sync_copy(tmin_h, tmin_v)
        pltpu.sync_copy(tdel_h, tdel_v)

        tmin_s = tmin_v[...] * MIN_VAR_ + MIN_MEAN_
        tmax_s = tmin_s + (jnp.abs(tdel_v[...]) * DELTA_VAR_ + DELTA_MEAN_)

        for i in range(STEPS):
            sl = pl.ds(i * L_, L_)
            idxv = idx_v[sl]
            i0 = idxv + idxv
            i1 = i0 + 1

            x0 = x0_v[sl]
            x1 = x1_v[sl]
            ee0 = e0_v[sl]
            ee1 = e1_v[sl]
            tm0 = tmin_s.at[i0].get(mode="promise_in_bounds")
            tm1 = tmin_s.at[i1].get(mode="promise_in_bounds")
            tx0 = tmax_s.at[i0].get(mode="promise_in_bounds")
            tx1 = tmax_s.at[i1].get(mode="promise_in_bounds")

            t1m0 = jnp.abs(x0) * MIN_VAR_ + MIN_MEAN_
            t1m1 = jnp.abs(x1) * MIN_VAR_ + MIN_MEAN_
            t1d0 = jnp.abs(ee0) * MIN_VAR_ + MIN_MEAN_
            t1d1 = jnp.abs(ee1) * MIN_VAR_ + MIN_MEAN_
            t1x0 = t1m0 + t1d0
            t1x1 = t1m1 + t1d1

            md0 = jnp.minimum(t1x0, tx0) - jnp.maximum(t1m0, tm0)
            md1 = jnp.minimum(t1x1, tx1) - jnp.maximum(t1m1, tm1)
            disjoint = (md0 <= 0.0) | (md1 <= 0.0)
            meetprod = jnp.maximum(md0, EPS_) * jnp.maximum(md1, EPS_)
            domiprod = t1d0 * t1d1
            joinprod = (jnp.maximum(t1x0, tx0) - jnp.minimum(t1m0, tm0)) * \
                       (jnp.maximum(t1x1, tx1) - jnp.minimum(t1m1, tm1))
            t2prod = (tx0 - tm0) * (tx1 - tm1)

            r = meetprod / domiprod
            pos_ov = -_vlog(r)
            neg_ov = -_vlog(jnp.maximum(1.0 - r, EPS_))
            q = jnp.maximum(1.0 - (domiprod + t2prod) / joinprod, EPS_) \
                * joinprod / domiprod
            pos_dis = -_vlog(q)

            pos_v[sl] = jnp.where(disjoint, pos_dis, pos_ov)
            neg_v[sl] = jnp.where(disjoint, 0.0, neg_ov)

        pltpu.sync_copy(pos_v, pos_h.at[pl.ds(base, BPW)])
        pltpu.sync_copy(neg_v, neg_h.at[pl.ds(base, BPW)])

    return pl.kernel(
        body,
        out_type=(jax.ShapeDtypeStruct((B,), jnp.float32),
                  jax.ShapeDtypeStruct((B,), jnp.float32)),
        mesh=mesh,
        scratch_types=[
            pltpu.VMEM((BPW,), jnp.float32),
            pltpu.VMEM((BPW,), jnp.float32),
            pltpu.VMEM((BPW,), jnp.float32),
            pltpu.VMEM((BPW,), jnp.float32),
            pltpu.VMEM((BPW,), jnp.int32),
            pltpu.VMEM((L_,), jnp.float32),
            pltpu.VMEM((L_,), jnp.float32),
            pltpu.VMEM((BPW,), jnp.float32),
            pltpu.VMEM((BPW,), jnp.float32),
        ],
    )


def kernel(t1x, t2_embed1, t2_embed2, min_feature_embed, delta_feature_embed):
    B = t1x.shape[0]
    call = _make_sc_call(B)
    return call(
        t1x.T.reshape(B * 2),
        t2_embed1.T.reshape(B * 2),
        t2_embed2.astype(jnp.int32),
        min_feature_embed.reshape(L_),
        delta_feature_embed.reshape(L_),
    )


# speedup vs baseline: 4.5268x; 1.0959x over previous
import jax
import jax.numpy as jnp
from jax import lax
from jax.experimental import pallas as pl
from jax.experimental.pallas import tpu as pltpu
from jax.experimental.pallas import tpu_sc as plsc

MIN_VAR_, MIN_MEAN_ = 5.5, 4.5
DELTA_VAR_, DELTA_MEAN_ = 0.95, 1.05
EPS_ = 1e-8
LN2_ = 0.6931471805599453
SQRT2_ = 1.4142135623730951
L_ = 16


def _vlog(x):
    bits = lax.bitcast_convert_type(x, jnp.int32)
    e = (bits >> 23) - 127
    m = lax.bitcast_convert_type((bits & 0x007FFFFF) | 0x3F800000, jnp.float32)
    big = m >= SQRT2_
    m = jnp.where(big, m * 0.5, m)
    e = e + jnp.where(big, 1, 0)
    s = (m - 1.0) / (m + 1.0)
    s2 = s * s
    p = s * (2.0 + s2 * (2.0 / 3.0 + s2 * (0.4 + s2 * (2.0 / 7.0 + s2 * (2.0 / 9.0)))))
    return e.astype(jnp.float32) * LN2_ + p


def _make_sc_call(B):
    info = plsc.get_sparse_core_info()
    NC, NS = info.num_cores, info.num_subcores
    NW = NC * NS
    assert B % (NW * L_) == 0
    BPW = B // NW
    STEPS = BPW // L_

    mesh = plsc.VectorSubcoreMesh(core_axis_name="c", subcore_axis_name="s")

    def body(t1_h, e_h, idx_h, tmin_h, tdel_h, pos_h, neg_h,
             x0_v, x1_v, e0_v, e1_v, idx_v, tmin_v, tdel_v, pos_v, neg_v):
        wid = lax.axis_index("s") * NC + lax.axis_index("c")
        base = wid * BPW

        pltpu.sync_copy(t1_h.at[pl.ds(base, BPW)], x0_v)
        pltpu.sync_copy(t1_h.at[pl.ds(B + base, BPW)], x1_v)
        pltpu.sync_copy(e_h.at[pl.ds(base, BPW)], e0_v)
        pltpu.sync_copy(e_h.at[pl.ds(B + base, BPW)], e1_v)
        pltpu.sync_copy(idx_h.at[pl.ds(base, BPW)], idx_v)
        pltpu.sync_copy(tmin_h, tmin_v)
        pltpu.sync_copy(tdel_h, tdel_v)

        tmin_s = tmin_v[...] * MIN_VAR_ + MIN_MEAN_
        tmax_s = tmin_s + (jnp.abs(tdel_v[...]) * DELTA_VAR_ + DELTA_MEAN_)

        @plsc.parallel_loop(0, BPW, step=L_, unroll=2)
        def _step(i):
            sl = pl.ds(i, L_)
            idxv = idx_v[sl]
            i0 = idxv + idxv
            i1 = i0 + 1

            x0 = x0_v[sl]
            x1 = x1_v[sl]
            ee0 = e0_v[sl]
            ee1 = e1_v[sl]
            tm0 = tmin_s.at[i0].get(mode="promise_in_bounds")
            tm1 = tmin_s.at[i1].get(mode="promise_in_bounds")
            tx0 = tmax_s.at[i0].get(mode="promise_in_bounds")
            tx1 = tmax_s.at[i1].get(mode="promise_in_bounds")

            t1m0 = jnp.abs(x0) * MIN_VAR_ + MIN_MEAN_
            t1m1 = jnp.abs(x1) * MIN_VAR_ + MIN_MEAN_
            t1d0 = jnp.abs(ee0) * MIN_VAR_ + MIN_MEAN_
            t1d1 = jnp.abs(ee1) * MIN_VAR_ + MIN_MEAN_
            t1x0 = t1m0 + t1d0
            t1x1 = t1m1 + t1d1

            md0 = jnp.minimum(t1x0, tx0) - jnp.maximum(t1m0, tm0)
            md1 = jnp.minimum(t1x1, tx1) - jnp.maximum(t1m1, tm1)
            disjoint = (md0 <= 0.0) | (md1 <= 0.0)
            meetprod = jnp.maximum(md0, EPS_) * jnp.maximum(md1, EPS_)
            domiprod = t1d0 * t1d1
            joinprod = (jnp.maximum(t1x0, tx0) - jnp.minimum(t1m0, tm0)) * \
                       (jnp.maximum(t1x1, tx1) - jnp.minimum(t1m1, tm1))
            t2prod = (tx0 - tm0) * (tx1 - tm1)

            r = meetprod / domiprod
            pos_ov = -_vlog(r)
            neg_ov = -_vlog(jnp.maximum(1.0 - r, EPS_))
            q = jnp.maximum(1.0 - (domiprod + t2prod) / joinprod, EPS_) \
                * joinprod / domiprod
            pos_dis = -_vlog(q)

            pos_v[sl] = jnp.where(disjoint, pos_dis, pos_ov)
            neg_v[sl] = jnp.where(disjoint, 0.0, neg_ov)

        pltpu.sync_copy(pos_v, pos_h.at[pl.ds(base, BPW)])
        pltpu.sync_copy(neg_v, neg_h.at[pl.ds(base, BPW)])

    return pl.kernel(
        body,
        out_type=(jax.ShapeDtypeStruct((B,), jnp.float32),
                  jax.ShapeDtypeStruct((B,), jnp.float32)),
        mesh=mesh,
        scratch_types=[
            pltpu.VMEM((BPW,), jnp.float32),
            pltpu.VMEM((BPW,), jnp.float32),
            pltpu.VMEM((BPW,), jnp.float32),
            pltpu.VMEM((BPW,), jnp.float32),
            pltpu.VMEM((BPW,), jnp.int32),
            pltpu.VMEM((L_,), jnp.float32),
            pltpu.VMEM((L_,), jnp.float32),
            pltpu.VMEM((BPW,), jnp.float32),
            pltpu.VMEM((BPW,), jnp.float32),
        ],
    )


def kernel(t1x, t2_embed1, t2_embed2, min_feature_embed, delta_feature_embed):
    B = t1x.shape[0]
    call = _make_sc_call(B)
    return call(
        t1x.T.reshape(B * 2),
        t2_embed1.T.reshape(B * 2),
        t2_embed2.astype(jnp.int32),
        min_feature_embed.reshape(L_),
        delta_feature_embed.reshape(L_),
    )
